# Initial kernel scaffold; baseline (speedup 1.0000x reference)
import jax
import jax.numpy as jnp
from jax.experimental import pallas as pl


def kernel(num_x, cat_x, tables, W1, b1, g1, be1, W2, b2, g2, be2, W3, b3, Ww, bw):
    raise NotImplementedError("write your pallas kernel here")


# speedup vs baseline: 7.6411x; 7.6411x over previous
import functools

import jax
import jax.numpy as jnp
from jax import lax
from jax.experimental import pallas as pl
from jax.experimental.pallas import tpu as pltpu
from jax.experimental.pallas import tpu_sc as plsc

B = 16384
NUM_NUMERIC = 13
NUM_CAT = 26
VOCAB = 100000
EMB = 16
H1 = 256
H2 = 128
EPS = 1e-5

TOT = B * NUM_CAT
NC, NS = 2, 16
NW = NC * NS
GPW = TOT // (NW * 128)
K = 8
SUP = GPW // K


@functools.cache
def _make_sc_gather():
    mesh = plsc.VectorSubcoreMesh(core_axis_name="c", subcore_axis_name="s")

    @functools.partial(
        pl.kernel,
        out_type=jax.ShapeDtypeStruct((TOT, EMB), jnp.float32),
        mesh=mesh,
        scratch_types=[
            pltpu.VMEM((GPW, 128), jnp.int32),
            pltpu.VMEM((K * 128, EMB), jnp.float32),
            pltpu.SemaphoreType.DMA,
        ],
        compiler_params=pltpu.CompilerParams(use_tc_tiling_on_sc=False),
    )
    def _sc_gather(tab_hbm, idx_hbm, out_hbm, idx_v, rows_v, sem):
        wid = lax.axis_index("s") * NC + lax.axis_index("c")
        base = wid * (GPW * 128)
        pltpu.sync_copy(idx_hbm.at[wid], idx_v)

        def sup_body(s, carry):
            cps = [
                pltpu.async_copy(
                    tab_hbm.at[idx_v.at[s * K + b]],
                    rows_v.at[pl.ds(b * 128, 128)],
                    sem,
                )
                for b in range(K)
            ]
            for cp in cps:
                cp.wait()
            pltpu.sync_copy(rows_v,
                            out_hbm.at[pl.ds(base + s * (K * 128), K * 128)])
            return carry

        lax.fori_loop(0, SUP, sup_body, 0)

    return _sc_gather


BS = 1024
NB = B // BS
_INV_B = 1.0 / B
_F32 = jnp.float32


def _a_body(nx_ref, em_ref, w1a_ref, w1b_ref, b1_ref, wwa_ref, wwb_ref,
            h1_ref, wide_ref, s1_ref, s2_ref):
    i = pl.program_id(0)
    nx = nx_ref[...]
    em = em_ref[...]
    h = (jnp.dot(nx, w1a_ref[...], preferred_element_type=_F32)
         + jnp.dot(em, w1b_ref[...], preferred_element_type=_F32)
         + b1_ref[...])
    h1_ref[...] = h
    wide_ref[...] = (jnp.sum(nx * wwa_ref[...], axis=1)
                     + jnp.sum(em * wwb_ref[...], axis=1))
    s1 = jnp.sum(h, axis=0, keepdims=True)
    s2 = jnp.sum(h * h, axis=0, keepdims=True)

    @pl.when(i == 0)
    def _():
        s1_ref[...] = s1
        s2_ref[...] = s2

    @pl.when(i > 0)
    def _():
        s1_ref[...] += s1
        s2_ref[...] += s2


_a_call = pl.pallas_call(
    _a_body,
    grid=(NB,),
    in_specs=[
        pl.BlockSpec((BS, NUM_NUMERIC), lambda i: (i, 0)),
        pl.BlockSpec((BS, NUM_CAT * EMB), lambda i: (i, 0)),
        pl.BlockSpec((NUM_NUMERIC, H1), lambda i: (0, 0)),
        pl.BlockSpec((NUM_CAT * EMB, H1), lambda i: (0, 0)),
        pl.BlockSpec((H1,), lambda i: (0,)),
        pl.BlockSpec((1, NUM_NUMERIC), lambda i: (0, 0)),
        pl.BlockSpec((1, NUM_CAT * EMB), lambda i: (0, 0)),
    ],
    out_specs=[
        pl.BlockSpec((BS, H1), lambda i: (i, 0)),
        pl.BlockSpec((BS,), lambda i: (i,)),
        pl.BlockSpec((1, H1), lambda i: (0, 0)),
        pl.BlockSpec((1, H1), lambda i: (0, 0)),
    ],
    out_shape=[
        jax.ShapeDtypeStruct((B, H1), _F32),
        jax.ShapeDtypeStruct((B,), _F32),
        jax.ShapeDtypeStruct((1, H1), _F32),
        jax.ShapeDtypeStruct((1, H1), _F32),
    ],
)


def _b_body(h1_ref, s1_ref, s2_ref, g1_ref, be1_ref, w2_ref, b2_ref,
            h2_ref, t1_ref, t2_ref):
    i = pl.program_id(0)
    mu = s1_ref[...] * _INV_B
    var = s2_ref[...] * _INV_B - mu * mu
    hn = jnp.maximum(
        g1_ref[...] * (h1_ref[...] - mu) * lax.rsqrt(var + EPS) + be1_ref[...],
        0.0)
    h2 = jnp.dot(hn, w2_ref[...], preferred_element_type=_F32) + b2_ref[...]
    h2_ref[...] = h2
    t1 = jnp.sum(h2, axis=0, keepdims=True)
    t2 = jnp.sum(h2 * h2, axis=0, keepdims=True)

    @pl.when(i == 0)
    def _():
        t1_ref[...] = t1
        t2_ref[...] = t2

    @pl.when(i > 0)
    def _():
        t1_ref[...] += t1
        t2_ref[...] += t2


_b_call = pl.pallas_call(
    _b_body,
    grid=(NB,),
    in_specs=[
        pl.BlockSpec((BS, H1), lambda i: (i, 0)),
        pl.BlockSpec((1, H1), lambda i: (0, 0)),
        pl.BlockSpec((1, H1), lambda i: (0, 0)),
        pl.BlockSpec((H1,), lambda i: (0,)),
        pl.BlockSpec((H1,), lambda i: (0,)),
        pl.BlockSpec((H1, H2), lambda i: (0, 0)),
        pl.BlockSpec((H2,), lambda i: (0,)),
    ],
    out_specs=[
        pl.BlockSpec((BS, H2), lambda i: (i, 0)),
        pl.BlockSpec((1, H2), lambda i: (0, 0)),
        pl.BlockSpec((1, H2), lambda i: (0, 0)),
    ],
    out_shape=[
        jax.ShapeDtypeStruct((B, H2), _F32),
        jax.ShapeDtypeStruct((1, H2), _F32),
        jax.ShapeDtypeStruct((1, H2), _F32),
    ],
)


def _c_body(h2_ref, t1_ref, t2_ref, g2_ref, be2_ref, w3_ref, wide_ref,
            b3w_ref, out_ref):
    mu = t1_ref[...] * _INV_B
    var = t2_ref[...] * _INV_B - mu * mu
    hn = jnp.maximum(
        g2_ref[...] * (h2_ref[...] - mu) * lax.rsqrt(var + EPS) + be2_ref[...],
        0.0)
    out_ref[...] = (jnp.sum(hn * w3_ref[...], axis=1) + wide_ref[...]
                    + b3w_ref[0, 0])


_c_call = pl.pallas_call(
    _c_body,
    grid=(NB,),
    in_specs=[
        pl.BlockSpec((BS, H2), lambda i: (i, 0)),
        pl.BlockSpec((1, H2), lambda i: (0, 0)),
        pl.BlockSpec((1, H2), lambda i: (0, 0)),
        pl.BlockSpec((H2,), lambda i: (0,)),
        pl.BlockSpec((H2,), lambda i: (0,)),
        pl.BlockSpec((1, H2), lambda i: (0, 0)),
        pl.BlockSpec((BS,), lambda i: (i,)),
        pl.BlockSpec(memory_space=pltpu.SMEM),
    ],
    out_specs=pl.BlockSpec((BS,), lambda i: (i,)),
    out_shape=jax.ShapeDtypeStruct((B,), _F32),
)


def kernel(num_x, cat_x, tables, W1, b1, g1, be1, W2, b2, g2, be2, W3, b3,
           Ww, bw):
    tab = tables.reshape(NUM_CAT * VOCAB, EMB)
    idx = (cat_x.astype(jnp.int32)
           + (jnp.arange(NUM_CAT, dtype=jnp.int32) * VOCAB)[None, :]
           ).reshape(NW, GPW, 128)
    em = _make_sc_gather()(tab, idx).reshape(B, NUM_CAT * EMB)
    w1a, w1b = W1[:NUM_NUMERIC], W1[NUM_NUMERIC:]
    wwa = Ww[:NUM_NUMERIC, 0][None, :]
    wwb = Ww[NUM_NUMERIC:, 0][None, :]
    w3row = W3[:, 0][None, :]
    b3w = (b3 + bw).reshape(1, 1)
    h1, wide, s1, s2 = _a_call(num_x, em, w1a, w1b, b1, wwa, wwb)
    h2, t1, t2 = _b_call(h1, s1, s2, g1, be1, W2, b2)
    return _c_call(h2, t1, t2, g2, be2, w3row, wide, b3w)
